# baseline (speedup 1.0000x reference)
import functools

import jax
import jax.numpy as jnp
from jax import lax
from jax.experimental import pallas as pl
from jax.experimental.pallas import tpu as pltpu
from jax.experimental.pallas import tpu_sc as plsc


def kernel(X):
    B, C, H, W = X.shape
    N = B * C
    HW = H * W
    Xr = X.reshape(N, HW)
    info = plsc.get_sparse_core_info()
    NC, NS, L = info.num_cores, info.num_subcores, info.num_lanes
    NW = NC * NS
    per_w = N // NW
    chunks = HW // L
    mesh = plsc.VectorSubcoreMesh(core_axis_name="c", subcore_axis_name="s")

    @functools.partial(
        pl.kernel,
        mesh=mesh,
        out_type=jax.ShapeDtypeStruct((N, HW), jnp.float32),
        scratch_types=[pltpu.VMEM((HW,), jnp.float32)],
    )
    def k(x_hbm, o_hbm, buf):
        wid = lax.axis_index("s") * NC + lax.axis_index("c")
        base = wid * per_w

        def per_map(m, carry):
            row = base + m
            pltpu.sync_copy(x_hbm.at[row], buf)

            def mx(i, acc):
                return jnp.maximum(acc, buf[pl.ds(i * L, L)])

            acc = lax.fori_loop(0, chunks, mx,
                                jnp.full((L,), -jnp.inf, jnp.float32))
            shift = 1
            while shift < L:
                idx = jnp.bitwise_xor(lax.iota(jnp.int32, L), shift)
                acc = jnp.maximum(
                    acc,
                    acc.at[idx].get(mode="promise_in_bounds",
                                    unique_indices=True))
                shift *= 2
            mvec = acc

            def sel(i, c):
                v = buf[pl.ds(i * L, L)]
                buf[pl.ds(i * L, L)] = jnp.where(v == mvec, v,
                                                 jnp.zeros_like(v))
                return c

            lax.fori_loop(0, chunks, sel, 0)
            pltpu.sync_copy(buf, o_hbm.at[row])
            return carry

        lax.fori_loop(0, per_w, per_map, 0)

    out = k(Xr)
    return out.reshape(B, C, H, W)


# speedup vs baseline: 10.4578x; 10.4578x over previous
import jax
import jax.numpy as jnp
from jax.experimental import pallas as pl
from jax.experimental.pallas import tpu as pltpu


_MAPS_PER_BLOCK = 64


def _wta_block(x_ref, o_ref):
    x = x_ref[...]
    m = jnp.max(x, axis=(1, 2), keepdims=True)
    o_ref[...] = jnp.where(x == m, x, jnp.zeros_like(x))


def kernel(X):
    B, C, H, W = X.shape
    N = B * C
    Xr = X.reshape(N, H, W)
    maps = _MAPS_PER_BLOCK
    if N % maps:
        maps = 1
    out = pl.pallas_call(
        _wta_block,
        grid=(N // maps,),
        in_specs=[pl.BlockSpec((maps, H, W), lambda i: (i, 0, 0))],
        out_specs=pl.BlockSpec((maps, H, W), lambda i: (i, 0, 0)),
        out_shape=jax.ShapeDtypeStruct((N, H, W), X.dtype),
        compiler_params=pltpu.CompilerParams(
            dimension_semantics=("parallel",),
        ),
    )(Xr)
    return out.reshape(B, C, H, W)
